# baseline (speedup 1.0000x reference)
import functools

import jax
import jax.numpy as jnp
from jax import lax
from jax.experimental import pallas as pl
from jax.experimental.pallas import tpu as pltpu
from jax.experimental.pallas import tpu_sc as plsc

N = 10000
E = 320000
D = 128
H1 = 64
H2 = 32
C = 4

NC = 2
NS = 16
NW = NC * NS
CHUNK = 192
NCH = -(-E // (NW * CHUNK))
EPAD = NW * CHUNK * NCH
NPAD = 10240
RPW = NPAD // NS
DCH = 128
DW = 16
BF = jnp.bfloat16


NB = 6
BLK = 1000
NBLK = N // BLK


def _make_sc_agg(d, gw, nb, with_deg):
    scratch = [
        pltpu.VMEM_SHARED((NPAD, d), BF),
        pltpu.VMEM((NCH, CHUNK), jnp.int32),
        pltpu.VMEM((NCH, CHUNK), jnp.int32),
        pltpu.VMEM((nb, CHUNK, gw), BF),
        pltpu.VMEM((DCH, d), BF),
        pltpu.SemaphoreType.DMA,
        pltpu.SemaphoreType.DMA,
        pltpu.SemaphoreType.DMA,
    ]
    out_type = jax.ShapeDtypeStruct((NC, NPAD, d), BF)
    if with_deg:
        out_type = (out_type,
                    jax.ShapeDtypeStruct((NC, NPAD, DW), BF))
        scratch += [
            pltpu.VMEM_SHARED((NPAD, DW), BF),
            pltpu.VMEM((CHUNK, DW), BF),
            pltpu.VMEM((DCH, DW), BF),
            pltpu.SemaphoreType.DMA,
            pltpu.SemaphoreType.DMA,
        ]
    mesh = plsc.VectorSubcoreMesh(core_axis_name="c", subcore_axis_name="s")

    def body(p_hbm, eg, *refs):
        if with_deg:
            (zc_hbm, on_hbm, zd_hbm, out_hbm, deg_hbm, acc_sh, src_v, dst_v,
             rows_v, zbuf, gsem, ssem, dsem, deg_sh, ones_v, zbuf8, osem,
             esem) = refs
        else:
            (zc_hbm, out_hbm, acc_sh, src_v, dst_v, rows_v, zbuf,
             gsem, ssem, dsem) = refs
        cid = lax.axis_index("c")
        sid = lax.axis_index("s")
        wid = sid * NC + cid

        pltpu.sync_copy(eg.at[0, wid], src_v)
        pltpu.sync_copy(eg.at[1, wid], dst_v)

        pltpu.sync_copy(zc_hbm, zbuf)
        base = sid * RPW
        for k in range(RPW // DCH):
            pltpu.sync_copy(zbuf, acc_sh.at[pl.ds(base + k * DCH, DCH)])
        if with_deg:
            pltpu.sync_copy(on_hbm, ones_v)
            pltpu.sync_copy(zd_hbm, zbuf8)
            for k in range(RPW // DCH):
                pltpu.sync_copy(zbuf8,
                                deg_sh.at[pl.ds(base + k * DCH, DCH)])
        plsc.subcore_barrier()

        for b in range(nb - 1):
            pltpu.async_copy(p_hbm.at[src_v.at[b]], rows_v.at[b], gsem)

        def step(j, carry):
            slot = lax.rem(j, nb)
            pslot = lax.rem(j + nb - 1, nb)
            pltpu.make_async_copy(p_hbm.at[src_v.at[j]],
                                  rows_v.at[slot], gsem).wait()
            pltpu.async_copy(rows_v.at[slot], acc_sh.at[dst_v.at[j]],
                             ssem, add=True)
            if with_deg:
                pltpu.async_copy(ones_v, deg_sh.at[dst_v.at[j]],
                                 osem, add=True)

                @pl.when(j >= 1)
                def _():
                    pltpu.make_async_copy(
                        ones_v, deg_sh.at[dst_v.at[0]], osem).wait()

            @pl.when(j >= 1)
            def _():
                pltpu.make_async_copy(
                    rows_v.at[0], acc_sh.at[dst_v.at[0]], ssem).wait()

            @pl.when(j + nb - 1 < NCH)
            def _():
                pltpu.async_copy(p_hbm.at[src_v.at[j + nb - 1]],
                                 rows_v.at[pslot], gsem)
            return carry
        lax.fori_loop(0, NCH, step, 0)
        pltpu.make_async_copy(rows_v.at[0], acc_sh.at[dst_v.at[0]],
                              ssem).wait()
        if with_deg:
            pltpu.make_async_copy(ones_v, deg_sh.at[dst_v.at[0]],
                                  osem).wait()
        plsc.subcore_barrier()

        def dv(slot):
            return rows_v.at[slot, pl.ds(0, DCH)]

        nd = RPW // DCH
        for k in range(nd):
            sl = pl.ds(base + k * DCH, DCH)
            slot = k % nb
            if k >= nb:
                pltpu.make_async_copy(dv(0), out_hbm.at[0, sl],
                                      dsem).wait()
            pltpu.sync_copy(acc_sh.at[sl], dv(slot))
            pltpu.async_copy(dv(slot), out_hbm.at[cid, sl], dsem)
            if with_deg:
                if k >= 1:
                    pltpu.make_async_copy(zbuf8, deg_hbm.at[0, sl],
                                          esem).wait()
                pltpu.sync_copy(deg_sh.at[sl], zbuf8)
                pltpu.async_copy(zbuf8, deg_hbm.at[cid, sl], esem)
        for k in range(min(nd, nb)):
            pltpu.make_async_copy(dv(0),
                                  out_hbm.at[0, pl.ds(0, DCH)], dsem).wait()
        if with_deg:
            pltpu.make_async_copy(zbuf8, deg_hbm.at[0, pl.ds(0, DCH)],
                                  esem).wait()

    return pl.kernel(
        body, out_type=out_type, mesh=mesh, scratch_types=scratch,
        compiler_params=pltpu.CompilerParams(use_tc_tiling_on_sc=False))


_sc_agg1 = _make_sc_agg(H1, H1, 3, True)
_sc_agg2 = _make_sc_agg(H2, H2, 6, False)


def _row_spec(d):
    return pl.BlockSpec((BLK, d), lambda i: (i, 0))


def _pair_spec(d):
    return pl.BlockSpec((2, BLK, d), lambda i: (0, i, 0))


def _full_spec(shape):
    nd = len(shape)
    return pl.BlockSpec(shape, lambda i, _n=nd: (0,) * _n)


def _proj1_body(x_ref, wl_ref, wr_ref, p_ref, r_ref):
    xb = x_ref[...]
    p = jnp.dot(xb, wl_ref[...], preferred_element_type=jnp.float32)
    p_ref[...] = p.astype(BF)
    r_ref[...] = jnp.dot(xb, wr_ref[...], preferred_element_type=jnp.float32)


def _proj1(x, wl, wr):
    return pl.pallas_call(
        _proj1_body,
        out_shape=(jax.ShapeDtypeStruct((N, H1), BF),
                   jax.ShapeDtypeStruct((N, H1), jnp.float32)),
    )(x, wl, wr)


def _comb1_body(s_ref, deg_ref, r1_ref, b1_ref, wl_ref,
                p2_ref, h1_ref):
    deg = jnp.maximum(deg_ref[0].astype(jnp.float32)
                      + deg_ref[1].astype(jnp.float32), 1.0)
    s = s_ref[0].astype(jnp.float32) + s_ref[1].astype(jnp.float32)
    agg = s * (1.0 / deg)[:, 0:1]
    h1 = jnp.maximum(agg[:N] + b1_ref[...] + r1_ref[...], 0.0)
    h1_ref[...] = h1
    p2 = jnp.dot(h1, wl_ref[...], preferred_element_type=jnp.float32)
    p2_ref[...] = p2.astype(BF)


def _comb1(s, deg, r1, b1, wl):
    return pl.pallas_call(
        _comb1_body,
        out_shape=(jax.ShapeDtypeStruct((N, H2), BF),
                   jax.ShapeDtypeStruct((N, H1), jnp.float32)),
    )(s, deg, r1, b1, wl)


def _r2k_body(h1_ref, wr_ref, r2_ref):
    r2_ref[...] = jnp.dot(h1_ref[...], wr_ref[...],
                          preferred_element_type=jnp.float32)


def _r2k(h1, wr):
    return pl.pallas_call(
        _r2k_body,
        out_shape=jax.ShapeDtypeStruct((N, H2), jnp.float32),
    )(h1, wr)


def _comb2_body(t_ref, deg_ref, r2_ref, b2_ref, wc_ref, bc_ref,
                h2_ref, z_ref):
    deg = jnp.maximum(deg_ref[0].astype(jnp.float32)
                      + deg_ref[1].astype(jnp.float32), 1.0)
    t = t_ref[0].astype(jnp.float32) + t_ref[1].astype(jnp.float32)
    agg = t * (1.0 / deg)[:, 0:1]
    h2 = jnp.maximum(agg[:N] + b2_ref[...] + r2_ref[...], 0.0)
    h2_ref[...] = h2
    z_ref[...] = (jnp.dot(h2, wc_ref[...], preferred_element_type=jnp.float32)
                  + bc_ref[...])


def _comb2(t, deg, r2, b2, wc, bc):
    return pl.pallas_call(
        _comb2_body,
        out_shape=(jax.ShapeDtypeStruct((N, H2), jnp.float32),
                   jax.ShapeDtypeStruct((N, C), jnp.float32)),
    )(t, deg, r2, b2, wc, bc)


def kernel(x, edge_index, W1_l, b1_l, W1_r, W2_l, b2_l, W2_r, Wc, bc):
    ei = edge_index.astype(jnp.int32)
    npe = EPAD - E
    pad_i = jnp.arange(npe, dtype=jnp.int32)
    pad_src = (pad_i * 97) % N
    pad_dst = N + pad_i % (NPAD - N)
    pad = jnp.stack([pad_src, pad_dst])
    eg = jnp.concatenate([ei, pad], axis=1).reshape(2, NW, NCH, CHUNK)

    zc1 = jnp.zeros((DCH, H1), BF)
    zc2 = jnp.zeros((DCH, H2), BF)
    on = jnp.ones((CHUNK, DW), BF)
    zd = jnp.zeros((DCH, DW), BF)

    p1, r1 = _proj1(x, W1_l, W1_r)
    s1, deg = _sc_agg1(p1, eg, zc1, on, zd)
    p2, h1 = _comb1(s1, deg, r1, b1_l.reshape(1, H1), W2_l)
    t2 = _sc_agg2(p2, eg, zc2)
    r2 = _r2k(h1, W2_r)
    h2, z = _comb2(t2, deg, r2, b2_l.reshape(1, H2), Wc,
                   bc.reshape(1, C))
    return (h2, z)


# speedup vs baseline: 1.0105x; 1.0105x over previous
import jax
import jax.numpy as jnp
from jax import lax
from jax.experimental import pallas as pl
from jax.experimental.pallas import tpu as pltpu
from jax.experimental.pallas import tpu_sc as plsc

N = 10000
E = 320000
D = 128
H1 = 64
H2 = 32
C = 4

NC = 2
NS = 16
NW = NC * NS
CHUNK = 192
NCH = -(-E // (NW * CHUNK))
EPAD = NW * CHUNK * NCH
NPAD = 10240
RPW = NPAD // NS
DCH = 128
DW = 16
BF = jnp.bfloat16


NB = 6
BLK = 1000
NBLK = N // BLK


def _make_sc_agg(d, gw, nb, with_deg):
    scratch = [
        pltpu.VMEM_SHARED((NPAD, d), BF),
        pltpu.VMEM((NCH, CHUNK), jnp.int32),
        pltpu.VMEM((NCH, CHUNK), jnp.int32),
        pltpu.VMEM((nb, CHUNK, gw), BF),
        pltpu.VMEM((DCH, d), BF),
        pltpu.SemaphoreType.DMA,
        pltpu.SemaphoreType.DMA,
        pltpu.SemaphoreType.DMA,
    ]
    out_type = jax.ShapeDtypeStruct((NC, NPAD, d), BF)
    if with_deg:
        out_type = (out_type,
                    jax.ShapeDtypeStruct((NC, NPAD, DW), BF))
        scratch += [
            pltpu.VMEM_SHARED((NPAD, DW), BF),
            pltpu.VMEM((CHUNK, DW), BF),
            pltpu.VMEM((DCH, DW), BF),
            pltpu.SemaphoreType.DMA,
            pltpu.SemaphoreType.DMA,
        ]
    mesh = plsc.VectorSubcoreMesh(core_axis_name="c", subcore_axis_name="s")

    def body(p_hbm, eg, *refs):
        if with_deg:
            (zc_hbm, on_hbm, zd_hbm, out_hbm, deg_hbm, acc_sh, src_v, dst_v,
             rows_v, zbuf, gsem, ssem, dsem, deg_sh, ones_v, zbuf8, osem,
             esem) = refs
        else:
            (zc_hbm, out_hbm, acc_sh, src_v, dst_v, rows_v, zbuf,
             gsem, ssem, dsem) = refs
        cid = lax.axis_index("c")
        sid = lax.axis_index("s")
        wid = sid * NC + cid

        pltpu.sync_copy(eg.at[0, wid], src_v)
        pltpu.sync_copy(eg.at[1, wid], dst_v)

        pltpu.sync_copy(zc_hbm, zbuf)
        base = sid * RPW
        for k in range(RPW // DCH):
            pltpu.sync_copy(zbuf, acc_sh.at[pl.ds(base + k * DCH, DCH)])
        if with_deg:
            pltpu.sync_copy(on_hbm, ones_v)
            pltpu.sync_copy(zd_hbm, zbuf8)
            for k in range(RPW // DCH):
                pltpu.sync_copy(zbuf8,
                                deg_sh.at[pl.ds(base + k * DCH, DCH)])
        plsc.subcore_barrier()

        for b in range(nb - 1):
            pltpu.async_copy(p_hbm.at[src_v.at[b]], rows_v.at[b], gsem)

        def step(j, carry):
            slot = lax.rem(j, nb)
            pslot = lax.rem(j + nb - 1, nb)
            pltpu.make_async_copy(p_hbm.at[src_v.at[j]],
                                  rows_v.at[slot], gsem).wait()
            pltpu.async_copy(rows_v.at[slot], acc_sh.at[dst_v.at[j]],
                             ssem, add=True)
            if with_deg:
                pltpu.async_copy(ones_v, deg_sh.at[dst_v.at[j]],
                                 osem, add=True)

                @pl.when(j >= 1)
                def _():
                    pltpu.make_async_copy(
                        ones_v, deg_sh.at[dst_v.at[0]], osem).wait()

            @pl.when(j >= 1)
            def _():
                pltpu.make_async_copy(
                    rows_v.at[0], acc_sh.at[dst_v.at[0]], ssem).wait()

            @pl.when(j + nb - 1 < NCH)
            def _():
                pltpu.async_copy(p_hbm.at[src_v.at[j + nb - 1]],
                                 rows_v.at[pslot], gsem)
            return carry
        lax.fori_loop(0, NCH, step, 0)
        pltpu.make_async_copy(rows_v.at[0], acc_sh.at[dst_v.at[0]],
                              ssem).wait()
        if with_deg:
            pltpu.make_async_copy(ones_v, deg_sh.at[dst_v.at[0]],
                                  osem).wait()
        plsc.subcore_barrier()

        def dv(slot):
            return rows_v.at[slot, pl.ds(0, DCH)]

        nd = RPW // DCH
        for k in range(nd):
            sl = pl.ds(base + k * DCH, DCH)
            slot = k % nb
            if k >= nb:
                pltpu.make_async_copy(dv(0), out_hbm.at[0, sl],
                                      dsem).wait()
            pltpu.sync_copy(acc_sh.at[sl], dv(slot))
            pltpu.async_copy(dv(slot), out_hbm.at[cid, sl], dsem)
            if with_deg:
                if k >= 1:
                    pltpu.make_async_copy(zbuf8, deg_hbm.at[0, sl],
                                          esem).wait()
                pltpu.sync_copy(deg_sh.at[sl], zbuf8)
                pltpu.async_copy(zbuf8, deg_hbm.at[cid, sl], esem)
        for k in range(min(nd, nb)):
            pltpu.make_async_copy(dv(0),
                                  out_hbm.at[0, pl.ds(0, DCH)], dsem).wait()
        if with_deg:
            pltpu.make_async_copy(zbuf8, deg_hbm.at[0, pl.ds(0, DCH)],
                                  esem).wait()

    return pl.kernel(
        body, out_type=out_type, mesh=mesh, scratch_types=scratch,
        compiler_params=pltpu.CompilerParams(use_tc_tiling_on_sc=False))


_sc_agg1 = _make_sc_agg(H1, H1, 3, True)
_sc_agg2 = _make_sc_agg(H2, H2, 6, False)


def _row_spec(d):
    return pl.BlockSpec((BLK, d), lambda i: (i, 0))


def _pair_spec(d):
    return pl.BlockSpec((2, BLK, d), lambda i: (0, i, 0))


def _full_spec(shape):
    nd = len(shape)
    return pl.BlockSpec(shape, lambda i, _n=nd: (0,) * _n)


def _proj1_body(x_ref, wl_ref, wr_ref, p_ref, r_ref):
    xb = x_ref[...]
    p = jnp.dot(xb, wl_ref[...], preferred_element_type=jnp.float32)
    p_ref[...] = p.astype(BF)
    r_ref[...] = jnp.dot(xb, wr_ref[...], preferred_element_type=jnp.float32)


def _proj1(x, wl, wr):
    return pl.pallas_call(
        _proj1_body,
        out_shape=(jax.ShapeDtypeStruct((N, H1), BF),
                   jax.ShapeDtypeStruct((N, H1), jnp.float32)),
    )(x, wl, wr)


def _comb1_body(s_ref, deg_ref, r1_ref, b1_ref, wl_ref, wr_ref,
                p2_ref, r2_ref):
    deg = jnp.maximum(deg_ref[0].astype(jnp.float32)
                      + deg_ref[1].astype(jnp.float32), 1.0)
    s = s_ref[0].astype(jnp.float32) + s_ref[1].astype(jnp.float32)
    agg = s * (1.0 / deg)[:, 0:1]
    h1 = jnp.maximum(agg[:N] + b1_ref[...] + r1_ref[...], 0.0)
    p2 = jnp.dot(h1, wl_ref[...], preferred_element_type=jnp.float32)
    p2_ref[...] = p2.astype(BF)
    r2_ref[...] = jnp.dot(h1, wr_ref[...], preferred_element_type=jnp.float32)


def _comb1(s, deg, r1, b1, wl, wr):
    return pl.pallas_call(
        _comb1_body,
        out_shape=(jax.ShapeDtypeStruct((N, H2), BF),
                   jax.ShapeDtypeStruct((N, H2), jnp.float32)),
    )(s, deg, r1, b1, wl, wr)


def _comb2_body(t_ref, deg_ref, r2_ref, b2_ref, wc_ref, bc_ref,
                h2_ref, z_ref):
    deg = jnp.maximum(deg_ref[0].astype(jnp.float32)
                      + deg_ref[1].astype(jnp.float32), 1.0)
    t = t_ref[0].astype(jnp.float32) + t_ref[1].astype(jnp.float32)
    agg = t * (1.0 / deg)[:, 0:1]
    h2 = jnp.maximum(agg[:N] + b2_ref[...] + r2_ref[...], 0.0)
    h2_ref[...] = h2
    z_ref[...] = (jnp.dot(h2, wc_ref[...], preferred_element_type=jnp.float32)
                  + bc_ref[...])


def _comb2(t, deg, r2, b2, wc, bc):
    return pl.pallas_call(
        _comb2_body,
        out_shape=(jax.ShapeDtypeStruct((N, H2), jnp.float32),
                   jax.ShapeDtypeStruct((N, C), jnp.float32)),
    )(t, deg, r2, b2, wc, bc)


def kernel(x, edge_index, W1_l, b1_l, W1_r, W2_l, b2_l, W2_r, Wc, bc):
    ei = edge_index.astype(jnp.int32)
    npe = EPAD - E
    pad_i = jnp.arange(npe, dtype=jnp.int32)
    pad_src = (pad_i * 97) % N
    pad_dst = N + pad_i % (NPAD - N)
    pad = jnp.stack([pad_src, pad_dst])
    eg = jnp.concatenate([ei, pad], axis=1).reshape(2, NW, NCH, CHUNK)

    zc1 = jnp.zeros((DCH, H1), BF)
    zc2 = jnp.zeros((DCH, H2), BF)
    on = jnp.ones((CHUNK, DW), BF)
    zd = jnp.zeros((DCH, DW), BF)

    p1, r1 = _proj1(x, W1_l, W1_r)
    s1, deg = _sc_agg1(p1, eg, zc1, on, zd)
    p2, r2 = _comb1(s1, deg, r1, b1_l.reshape(1, H1), W2_l, W2_r)
    t2 = _sc_agg2(p2, eg, zc2)
    h2, z = _comb2(t2, deg, r2, b2_l.reshape(1, H2), Wc,
                   bc.reshape(1, C))
    return (h2, z)
